# baseline (speedup 1.0000x reference)
import functools
import math

import jax
import jax.numpy as jnp
from jax import lax
from jax.experimental import pallas as pl
from jax.experimental.pallas import tpu as pltpu
from jax.experimental.pallas import tpu_sc as plsc

MAXLEN = 200
EMB = 128
NC, NS = 2, 16
NW = NC * NS
N = 1024 * MAXLEN
PER_W = N // NW
CH = 64
NCH = PER_W // CH
LANES = 16


def _rel_body(o_ref):
    pi = lax.broadcasted_iota(jnp.int32, (MAXLEN, EMB), 0)
    ci = lax.broadcasted_iota(jnp.int32, (MAXLEN, EMB), 1)
    p = pi.astype(jnp.float32)
    c = ci.astype(jnp.float32)
    div = jnp.exp(c * (math.log(10000.0) / (EMB // 2)))
    arg = p / div
    o_ref[...] = jnp.where(ci < EMB // 2, jnp.sin(arg), jnp.cos(arg))


@functools.cache
def _make_sc_gather():
    mesh = plsc.VectorSubcoreMesh(
        core_axis_name="c", subcore_axis_name="s",
        num_cores=NC, num_subcores=NS)

    buf = pltpu.VMEM((CH * EMB,), jnp.float32)
    tab = pltpu.VMEM((MAXLEN * EMB,), jnp.float32)
    dma = pltpu.SemaphoreType.DMA

    @functools.partial(
        pl.kernel,
        out_type=(
            jax.ShapeDtypeStruct((N * EMB,), jnp.float32),
            jax.ShapeDtypeStruct((N * EMB,), jnp.float32),
        ),
        mesh=mesh,
        compiler_params=pltpu.CompilerParams(needs_layout_passes=False),
        scratch_types=(
            [pltpu.VMEM((NCH, CH), jnp.int32), tab, tab]
            + [buf] * 4 + [dma] * 4
        ),
    )
    def sc_gather(table_hbm, rel_hbm, idx_hbm, out_a, out_r,
                  idx_v, tab_a, tab_r,
                  ba0, ba1, br0, br1,
                  wsa0, wsa1, wsr0, wsr1):
        ba = (ba0, ba1)
        br = (br0, br1)
        wsa = (wsa0, wsa1)
        wsr = (wsr0, wsr1)

        wid = lax.axis_index("s") * NC + lax.axis_index("c")
        base = wid * (PER_W * EMB)
        pltpu.sync_copy(table_hbm, tab_a)
        pltpu.sync_copy(rel_hbm, tab_r)
        pltpu.sync_copy(idx_hbm.at[wid], idx_v)

        def wstart(j, s):
            dst = pl.ds(base + j * (CH * EMB), CH * EMB)
            pltpu.async_copy(ba[s], out_a.at[dst], wsa[s])
            pltpu.async_copy(br[s], out_r.at[dst], wsr[s])

        def wwait(s):
            dst = pl.ds(base, CH * EMB)
            pltpu.make_async_copy(ba[s], out_a.at[dst], wsa[s]).wait()
            pltpu.make_async_copy(br[s], out_r.at[dst], wsr[s]).wait()

        @pl.loop(0, NCH, step=2)
        def _outer(j0):
            for s in range(2):
                j = j0 + s

                @pl.when(j >= 2)
                def _():
                    wwait(s)

                for g in range(CH // LANES):
                    rowb = idx_v[j, pl.ds(g * LANES, LANES)] * EMB
                    for i in range(LANES):
                        r = rowb[i]
                        d0 = (g * LANES + i) * EMB
                        va = [tab_a[pl.ds(r + k, LANES)]
                              for k in range(0, EMB, LANES)]
                        vr = [tab_r[pl.ds(r + k, LANES)]
                              for k in range(0, EMB, LANES)]
                        for n, k in enumerate(range(0, EMB, LANES)):
                            ba[s][pl.ds(d0 + k, LANES)] = va[n]
                            br[s][pl.ds(d0 + k, LANES)] = vr[n]

                wstart(j, s)

        wwait(0)
        wwait(1)

    return sc_gather


def kernel(pos, table):
    rel_tab = pl.pallas_call(
        _rel_body,
        out_shape=jax.ShapeDtypeStruct((MAXLEN, EMB), jnp.float32),
    )()
    idx = pos.reshape(NW, NCH, CH)
    out_a, out_r = _make_sc_gather()(
        table.reshape(MAXLEN * EMB), rel_tab.reshape(MAXLEN * EMB), idx)
    b, l = pos.shape
    return out_a.reshape(b, l, EMB), out_r.reshape(b, l, EMB)


# speedup vs baseline: 1.0069x; 1.0069x over previous
import functools

import jax
import jax.numpy as jnp
from jax import lax
from jax.experimental import pallas as pl
from jax.experimental.pallas import tpu as pltpu
from jax.experimental.pallas import tpu_sc as plsc

MAXLEN = 200
EMB = 128
NC, NS = 2, 16
NW = NC * NS
N = 1024 * MAXLEN
PER_W = N // NW
CH = 80
NCH = PER_W // CH
NBUF = 4

BR = 8
BC = 256
NROW = N // BC


def _rel_body(pos_ref, cst_ref, o_ref):
    p3 = pos_ref[...].astype(jnp.float32)[:, :, None]
    div = cst_ref[0, :][None, None, :]
    off = cst_ref[1, :][None, None, :]
    o_ref[...] = jnp.sin(p3 / div + off)


@functools.partial(jax.jit, static_argnames=())
def _tc_rel(pos_r, cst):
    return pl.pallas_call(
        _rel_body,
        grid=(NROW // BR,),
        in_specs=[
            pl.BlockSpec((BR, BC), lambda i: (i, 0)),
            pl.BlockSpec((2, EMB), lambda i: (0, 0)),
        ],
        out_specs=pl.BlockSpec((BR, BC, EMB), lambda i: (i, 0, 0)),
        out_shape=jax.ShapeDtypeStruct((NROW, BC, EMB), jnp.float32),
    )(pos_r, cst)


@functools.cache
def _make_sc_gather():
    mesh = plsc.VectorSubcoreMesh(
        core_axis_name="c", subcore_axis_name="s",
        num_cores=NC, num_subcores=NS)

    row_buf = pltpu.VMEM((CH, EMB), jnp.float32)
    dma = pltpu.SemaphoreType.DMA

    @functools.partial(
        pl.kernel,
        out_type=jax.ShapeDtypeStruct((N, EMB), jnp.float32),
        mesh=mesh,
        scratch_types=(
            [pltpu.VMEM((NCH, CH), jnp.int32)]
            + [row_buf] * NBUF + [dma] * (2 * NBUF)
        ),
    )
    def sc_gather(table_hbm, idx_hbm, out_a,
                  idx_v,
                  ba0, ba1, ba2, ba3,
                  gsa0, gsa1, gsa2, gsa3,
                  wsa0, wsa1, wsa2, wsa3):
        ba = (ba0, ba1, ba2, ba3)
        gsa = (gsa0, gsa1, gsa2, gsa3)
        wsa = (wsa0, wsa1, wsa2, wsa3)

        wid = lax.axis_index("s") * NC + lax.axis_index("c")
        base = wid * PER_W
        pltpu.sync_copy(idx_hbm.at[wid], idx_v)

        def gstart(j, s):
            pltpu.async_copy(table_hbm.at[idx_v.at[j]], ba[s], gsa[s])

        def gwait(s):
            pltpu.make_async_copy(table_hbm.at[idx_v.at[0]], ba[s], gsa[s]).wait()

        def wstart(j, s):
            pltpu.async_copy(ba[s], out_a.at[pl.ds(base + j * CH, CH)], wsa[s])

        def wwait(s):
            pltpu.make_async_copy(ba[s], out_a.at[pl.ds(base, CH)], wsa[s]).wait()

        gstart(0, 0)
        gstart(1, 1)

        @pl.loop(0, NCH, step=NBUF)
        def _outer(i0):
            for b in range(NBUF):
                i = i0 + b
                s = b
                s2 = (b + 2) % NBUF
                gwait(s)
                wstart(i, s)

                @pl.when(i >= 2)
                def _():
                    wwait(s2)

                @pl.when(i + 2 < NCH)
                def _():
                    gstart(i + 2, s2)

        wwait(2)
        wwait(3)

    return sc_gather


def kernel(pos, table):
    b, l = pos.shape
    idx = pos.reshape(NW, NCH, CH)
    out_a = _make_sc_gather()(table, idx)

    div = jnp.power(10000.0, jnp.arange(0, 2 * EMB, 2, dtype=jnp.float32) / EMB)
    off = jnp.where(jnp.arange(EMB) < EMB // 2, 0.0,
                    jnp.float32(jnp.pi / 2)).astype(jnp.float32)
    cst = jnp.stack([div, off])
    out_r = _tc_rel(pos.reshape(NROW, BC), cst)

    return out_a.reshape(b, l, EMB), out_r.reshape(b, l, EMB)


# speedup vs baseline: 1.3916x; 1.3820x over previous
import functools
import math

import jax
import jax.numpy as jnp
from jax import lax
from jax.experimental import pallas as pl
from jax.experimental.pallas import tpu as pltpu
from jax.experimental.pallas import tpu_sc as plsc

MAXLEN = 200
EMB = 128
NC, NS = 2, 16
NW = NC * NS
N = 1024 * MAXLEN
PER_W = N // NW
CH = 80
NCH = PER_W // CH
NBUF = 4

BR = 8
BC = 256
NROW = N // BC


def _rel_body(pos_ref, cst_ref, o_ref):
    p3 = pos_ref[...].astype(jnp.float32)[:, :, None]
    div = cst_ref[0, :][None, None, :]
    off = cst_ref[1, :][None, None, :]
    x = p3 / div + off
    y = x * jnp.float32(1.0 / math.pi)
    n = jnp.round(y)
    t = (y - n) * jnp.float32(math.pi)
    t2 = t * t
    s = t * (1.0 + t2 * (jnp.float32(-1.0 / 6.0)
             + t2 * (jnp.float32(1.0 / 120.0)
             + t2 * (jnp.float32(-1.0 / 5040.0)
             + t2 * jnp.float32(1.0 / 362880.0)))))
    odd = (n.astype(jnp.int32) & 1) == 1
    o_ref[...] = jnp.where(odd, -s, s)


@functools.partial(jax.jit, static_argnames=())
def _tc_rel(pos_r, cst):
    return pl.pallas_call(
        _rel_body,
        grid=(NROW // BR,),
        in_specs=[
            pl.BlockSpec((BR, BC), lambda i: (i, 0)),
            pl.BlockSpec((2, EMB), lambda i: (0, 0)),
        ],
        out_specs=pl.BlockSpec((BR, BC, EMB), lambda i: (i, 0, 0)),
        out_shape=jax.ShapeDtypeStruct((NROW, BC, EMB), jnp.float32),
    )(pos_r, cst)


@functools.cache
def _make_sc_gather():
    mesh = plsc.VectorSubcoreMesh(
        core_axis_name="c", subcore_axis_name="s",
        num_cores=NC, num_subcores=NS)

    row_buf = pltpu.VMEM((CH, EMB), jnp.float32)
    dma = pltpu.SemaphoreType.DMA

    @functools.partial(
        pl.kernel,
        out_type=jax.ShapeDtypeStruct((N, EMB), jnp.float32),
        mesh=mesh,
        scratch_types=(
            [pltpu.VMEM((NCH, CH), jnp.int32)]
            + [row_buf] * NBUF + [dma] * (2 * NBUF)
        ),
    )
    def sc_gather(table_hbm, idx_hbm, out_a,
                  idx_v,
                  ba0, ba1, ba2, ba3,
                  gsa0, gsa1, gsa2, gsa3,
                  wsa0, wsa1, wsa2, wsa3):
        ba = (ba0, ba1, ba2, ba3)
        gsa = (gsa0, gsa1, gsa2, gsa3)
        wsa = (wsa0, wsa1, wsa2, wsa3)

        wid = lax.axis_index("s") * NC + lax.axis_index("c")
        base = wid * PER_W
        pltpu.sync_copy(idx_hbm.at[wid], idx_v)

        def gstart(j, s):
            pltpu.async_copy(table_hbm.at[idx_v.at[j]], ba[s], gsa[s])

        def gwait(s):
            pltpu.make_async_copy(table_hbm.at[idx_v.at[0]], ba[s], gsa[s]).wait()

        def wstart(j, s):
            pltpu.async_copy(ba[s], out_a.at[pl.ds(base + j * CH, CH)], wsa[s])

        def wwait(s):
            pltpu.make_async_copy(ba[s], out_a.at[pl.ds(base, CH)], wsa[s]).wait()

        gstart(0, 0)
        gstart(1, 1)

        @pl.loop(0, NCH, step=NBUF)
        def _outer(i0):
            for b in range(NBUF):
                i = i0 + b
                s = b
                s2 = (b + 2) % NBUF
                gwait(s)
                wstart(i, s)

                @pl.when(i >= 2)
                def _():
                    wwait(s2)

                @pl.when(i + 2 < NCH)
                def _():
                    gstart(i + 2, s2)

        wwait(2)
        wwait(3)

    return sc_gather


def kernel(pos, table):
    b, l = pos.shape
    idx = pos.reshape(NW, NCH, CH)
    out_a = _make_sc_gather()(table, idx)

    div = jnp.power(10000.0, jnp.arange(0, 2 * EMB, 2, dtype=jnp.float32) / EMB)
    off = jnp.where(jnp.arange(EMB) < EMB // 2, 0.0,
                    jnp.float32(jnp.pi / 2)).astype(jnp.float32)
    cst = jnp.stack([div, off])
    out_r = _tc_rel(pos.reshape(NROW, BC), cst)

    return out_a.reshape(b, l, EMB), out_r.reshape(b, l, EMB)
